# baseline (speedup 1.0000x reference)
import jax
import jax.numpy as jnp
from jax.experimental import pallas as pl

N_RATE = 6
N_GENRE = 25
N_DIRECTOR = 2186
N_YEAR = 81
EMB = 32
D = 2 + N_GENRE + N_DIRECTOR
DPAD = 2304
BT = 1024
NC = 3
CW = DPAD // NC


def _tile_kernel(*refs):
    x_refs = refs[:NC]
    w_refs = refs[NC:2 * NC]
    w_rate_ref, w_year_ref, out_ref = refs[2 * NC:]

    acc = None
    for c in range(NC):
        xf = x_refs[c][...].astype(jnp.float32)
        part = jax.lax.dot_general(
            xf, w_refs[c][...],
            (((1,), (0,)), ((), ())),
            preferred_element_type=jnp.float32,
        )
        acc = part if acc is None else acc + part
    gd = jax.nn.sigmoid(acc)

    rate_idx = x_refs[0][:, 0:1]
    year_idx = x_refs[0][:, 1:2]
    oh_rate = (rate_idx == jax.lax.broadcasted_iota(jnp.int32, (1, N_RATE), 1)
               ).astype(jnp.float32)
    oh_year = (year_idx == jax.lax.broadcasted_iota(jnp.int32, (1, N_YEAR), 1)
               ).astype(jnp.float32)
    rate_emb = jax.lax.dot_general(
        oh_rate, w_rate_ref[...], (((1,), (0,)), ((), ())),
        preferred_element_type=jnp.float32)
    year_emb = jax.lax.dot_general(
        oh_year, w_year_ref[...], (((1,), (0,)), ((), ())),
        preferred_element_type=jnp.float32)

    out_ref[...] = jnp.concatenate([rate_emb, year_emb, gd], axis=1)


def kernel(x, W_rate, W_year, W_genre, W_director):
    B = x.shape[0]
    W_big = jnp.zeros((DPAD, 2 * EMB), jnp.float32)
    W_big = W_big.at[2:2 + N_GENRE, 0:EMB].set(W_genre)
    W_big = W_big.at[2 + N_GENRE:D, EMB:].set(W_director)
    w_chunks = [W_big[c * CW:(c + 1) * CW] for c in range(NC)]

    x_specs = [
        pl.BlockSpec((BT, CW), lambda i, c=c: (i, c))
        for c in range(NC)
    ]
    w_specs = [pl.BlockSpec((CW, 2 * EMB), lambda i: (0, 0)) for _ in range(NC)]

    return pl.pallas_call(
        _tile_kernel,
        grid=(B // BT,),
        in_specs=x_specs + w_specs + [
            pl.BlockSpec((N_RATE, EMB), lambda i: (0, 0)),
            pl.BlockSpec((N_YEAR, EMB), lambda i: (0, 0)),
        ],
        out_specs=pl.BlockSpec((BT, 4 * EMB), lambda i: (i, 0)),
        out_shape=jax.ShapeDtypeStruct((B, 4 * EMB), jnp.float32),
    )(*([x] * NC + w_chunks + [W_rate, W_year]))
